# baseline (speedup 1.0000x reference)
import functools

import jax
import jax.numpy as jnp
from jax import lax
from jax.experimental import pallas as pl
from jax.experimental.pallas import tpu as pltpu
from jax.experimental.pallas import tpu_sc as plsc

B, L, D = 4096, 200, 128
LANES = 16
PE_ROWS = 208
NB = 8
LSEG = 40
NSEG = L // LSEG


def kernel(inputs, pos_embedding):
    info = plsc.get_sparse_core_info()
    nc, ns = info.num_cores, info.num_subcores
    nw = nc * ns
    rows = B // nw
    groups = rows // NB
    units = groups * NSEG

    mesh = plsc.VectorSubcoreMesh(core_axis_name="c", subcore_axis_name="s")

    @functools.partial(
        pl.kernel,
        mesh=mesh,
        out_type=jax.ShapeDtypeStruct((B, L, D), jnp.float32),
        scratch_types=[
            pltpu.VMEM((PE_ROWS, D), jnp.float32),
            pltpu.VMEM((NB, LSEG, D), jnp.float32),
            pltpu.VMEM((NB, LSEG, D), jnp.float32),
            pltpu.SemaphoreType.DMA,
            pltpu.SemaphoreType.DMA,
        ],
    )
    def sc_add(in_hbm, pe_hbm, out_hbm, pe_v, buf0, buf1, s0, s1):
        c = lax.axis_index("c")
        s = lax.axis_index("s")
        wid = s * nc + c
        base = wid * rows

        pltpu.sync_copy(pe_hbm.at[pl.ds(0, PE_ROWS)], pe_v)

        def unit_slice(u):
            g = u // NSEG
            seg = u - g * NSEG
            return base + g * NB, seg * LSEG

        def in_slc(u):
            r, l0 = unit_slice(u)
            return in_hbm.at[pl.ds(r, NB), pl.ds(l0, LSEG), :]

        def out_slc(u):
            r, l0 = unit_slice(u)
            return out_hbm.at[pl.ds(r, NB), pl.ds(l0, LSEG), :]

        def add_pe(buf, u):
            _, l0 = unit_slice(u)

            def body(l, carry):
                for j in range(D // LANES):
                    sl = pl.ds(j * LANES, LANES)
                    pe = pe_v[l0 + l + 1, sl]
                    for r in range(NB):
                        buf[r, l, sl] = buf[r, l, sl] + pe
                return carry

            lax.fori_loop(0, LSEG, body, 0)

        pltpu.async_copy(in_slc(0), buf0, s0)
        pltpu.async_copy(in_slc(1), buf1, s1)

        def step(t, carry):
            u0 = 2 * t
            u1 = u0 + 1
            pltpu.make_async_copy(in_slc(u0), buf0, s0).wait()
            add_pe(buf0, u0)
            pltpu.async_copy(buf0, out_slc(u0), s0)
            pltpu.make_async_copy(in_slc(u1), buf1, s1).wait()
            add_pe(buf1, u1)
            pltpu.async_copy(buf1, out_slc(u1), s1)
            pltpu.make_async_copy(buf0, out_slc(u0), s0).wait()
            pltpu.async_copy(in_slc(u0 + 2), buf0, s0)
            pltpu.make_async_copy(buf1, out_slc(u1), s1).wait()
            pltpu.async_copy(in_slc(u1 + 2), buf1, s1)
            return carry

        lax.fori_loop(0, units // 2 - 1, step, 0)

        u0 = units - 2
        u1 = units - 1
        pltpu.make_async_copy(in_slc(u0), buf0, s0).wait()
        add_pe(buf0, u0)
        pltpu.async_copy(buf0, out_slc(u0), s0)
        pltpu.make_async_copy(in_slc(u1), buf1, s1).wait()
        add_pe(buf1, u1)
        pltpu.async_copy(buf1, out_slc(u1), s1)
        pltpu.make_async_copy(buf0, out_slc(u0), s0).wait()
        pltpu.make_async_copy(buf1, out_slc(u1), s1).wait()

    return sc_add(inputs, pos_embedding)


# speedup vs baseline: 1.0942x; 1.0942x over previous
import functools

import jax
import jax.numpy as jnp
from jax import lax
from jax.experimental import pallas as pl
from jax.experimental.pallas import tpu as pltpu
from jax.experimental.pallas import tpu_sc as plsc

B, L, D = 4096, 200, 128
LANES = 16
NB = 4
LSEG = 40
NPHASE = L // LSEG
NBUF = 4


def kernel(inputs, pos_embedding):
    info = plsc.get_sparse_core_info()
    nc, ns = info.num_cores, info.num_subcores
    nw = nc * ns
    rows = B // nw
    groups = rows // NB

    mesh = plsc.VectorSubcoreMesh(core_axis_name="c", subcore_axis_name="s")

    @functools.partial(
        pl.kernel,
        mesh=mesh,
        out_type=jax.ShapeDtypeStruct((B, L, D), jnp.float32),
        scratch_types=[
            pltpu.VMEM((LSEG + 8, D), jnp.float32),
            pltpu.VMEM((NB, LSEG, D), jnp.float32),
            pltpu.VMEM((NB, LSEG, D), jnp.float32),
            pltpu.VMEM((NB, LSEG, D), jnp.float32),
            pltpu.VMEM((NB, LSEG, D), jnp.float32),
            pltpu.SemaphoreType.DMA,
            pltpu.SemaphoreType.DMA,
            pltpu.SemaphoreType.DMA,
            pltpu.SemaphoreType.DMA,
        ],
    )
    def sc_add(in_hbm, pe_hbm, out_hbm, pe_v, b0, b1, b2, b3, s0, s1, s2, s3):
        c = lax.axis_index("c")
        s = lax.axis_index("s")
        wid = s * nc + c
        base = wid * rows
        bufs = (b0, b1, b2, b3)
        sems = (s0, s1, s2, s3)

        for p in range(NPHASE):
            l0 = p * LSEG
            pltpu.sync_copy(pe_hbm.at[pl.ds(l0, LSEG + 8)], pe_v)

            def in_slc(u):
                r = base + u * NB
                return in_hbm.at[pl.ds(r, NB), pl.ds(l0, LSEG), :]

            def out_slc(u):
                r = base + u * NB
                return out_hbm.at[pl.ds(r, NB), pl.ds(l0, LSEG), :]

            def add_pe(buf):
                def body(l, carry):
                    for j in range(D // LANES):
                        sl = pl.ds(j * LANES, LANES)
                        pe = pe_v[l + 1, sl]
                        for r in range(NB):
                            buf[r, l, sl] = buf[r, l, sl] + pe
                    return carry
                lax.fori_loop(0, LSEG, body, 0)

            for k in range(NBUF):
                pltpu.async_copy(in_slc(k), bufs[k], sems[k])

            def step(t, carry):
                for k in range(NBUF):
                    u = NBUF * t + k
                    pltpu.make_async_copy(in_slc(u), bufs[k], sems[k]).wait()
                    add_pe(bufs[k])
                    pltpu.async_copy(bufs[k], out_slc(u), sems[k])
                for k in range(NBUF):
                    u = NBUF * t + k
                    pltpu.make_async_copy(bufs[k], out_slc(u), sems[k]).wait()
                    pltpu.async_copy(in_slc(u + NBUF), bufs[k], sems[k])
                return carry

            lax.fori_loop(0, groups // NBUF - 1, step, 0)

            for k in range(NBUF):
                u = groups - NBUF + k
                pltpu.make_async_copy(in_slc(u), bufs[k], sems[k]).wait()
                add_pe(bufs[k])
                pltpu.async_copy(bufs[k], out_slc(u), sems[k])
            for k in range(NBUF):
                u = groups - NBUF + k
                pltpu.make_async_copy(bufs[k], out_slc(u), sems[k]).wait()

    return sc_add(inputs, pos_embedding)


# speedup vs baseline: 1.1442x; 1.0457x over previous
import functools

import jax
import jax.numpy as jnp
from jax import lax
from jax.experimental import pallas as pl
from jax.experimental.pallas import tpu as pltpu
from jax.experimental.pallas import tpu_sc as plsc

B, L, D = 4096, 200, 128
LANES = 16
NB = 4
LSEG = 40
NPHASE = L // LSEG
NBUF = 2


def kernel(inputs, pos_embedding):
    info = plsc.get_sparse_core_info()
    nc, ns = info.num_cores, info.num_subcores
    nw = nc * ns
    rows = B // nw
    groups = rows // NB

    mesh = plsc.VectorSubcoreMesh(core_axis_name="c", subcore_axis_name="s")

    @functools.partial(
        pl.kernel,
        mesh=mesh,
        out_type=jax.ShapeDtypeStruct((B, L, D), jnp.float32),
        scratch_types=[
            pltpu.VMEM((LSEG + 8, D), jnp.float32),
            pltpu.VMEM((NB, LSEG, D), jnp.float32),
            pltpu.VMEM((NB, LSEG, D), jnp.float32),
            pltpu.VMEM((NB, LSEG, D), jnp.float32),
            pltpu.VMEM((NB, LSEG, D), jnp.float32),
            pltpu.SemaphoreType.DMA,
            pltpu.SemaphoreType.DMA,
            pltpu.SemaphoreType.DMA,
            pltpu.SemaphoreType.DMA,
        ],
    )
    def sc_add(in_hbm, pe_hbm, out_hbm, pe_v, i0, i1, o0, o1,
               si0, si1, so0, so1):
        c = lax.axis_index("c")
        s = lax.axis_index("s")
        wid = s * nc + c
        base = wid * rows
        ibufs = (i0, i1)
        obufs = (o0, o1)
        isems = (si0, si1)
        osems = (so0, so1)

        for p in range(NPHASE):
            l0 = p * LSEG
            pltpu.sync_copy(pe_hbm.at[pl.ds(l0, LSEG + 8)], pe_v)

            def in_slc(u):
                r = base + u * NB
                return in_hbm.at[pl.ds(r, NB), pl.ds(l0, LSEG), :]

            def out_slc(u):
                r = base + u * NB
                return out_hbm.at[pl.ds(r, NB), pl.ds(l0, LSEG), :]

            def add_pe(ibuf, obuf):
                def body(l, carry):
                    for j in range(D // LANES):
                        sl = pl.ds(j * LANES, LANES)
                        pe = pe_v[l + 1, sl]
                        for r in range(NB):
                            obuf[r, l, sl] = ibuf[r, l, sl] + pe
                    return carry
                lax.fori_loop(0, LSEG, body, 0)

            for k in range(NBUF):
                pltpu.async_copy(in_slc(k), ibufs[k], isems[k])

            for k in range(NBUF):
                pltpu.make_async_copy(in_slc(k), ibufs[k], isems[k]).wait()
                add_pe(ibufs[k], obufs[k])
                pltpu.async_copy(obufs[k], out_slc(k), osems[k])
                pltpu.async_copy(in_slc(k + NBUF), ibufs[k], isems[k])

            def step(t, carry):
                for k in range(NBUF):
                    u = NBUF * t + k
                    pltpu.make_async_copy(in_slc(u), ibufs[k], isems[k]).wait()
                    pltpu.make_async_copy(
                        obufs[k], out_slc(u - NBUF), osems[k]).wait()
                    add_pe(ibufs[k], obufs[k])
                    pltpu.async_copy(obufs[k], out_slc(u), osems[k])
                    pltpu.async_copy(in_slc(u + NBUF), ibufs[k], isems[k])
                return carry

            lax.fori_loop(1, groups // NBUF - 1, step, 0)

            for k in range(NBUF):
                u = groups - NBUF + k
                pltpu.make_async_copy(in_slc(u), ibufs[k], isems[k]).wait()
                pltpu.make_async_copy(
                    obufs[k], out_slc(u - NBUF), osems[k]).wait()
                add_pe(ibufs[k], obufs[k])
                pltpu.async_copy(obufs[k], out_slc(u), osems[k])
            for k in range(NBUF):
                u = groups - NBUF + k
                pltpu.make_async_copy(obufs[k], out_slc(u), osems[k]).wait()

    return sc_add(inputs, pos_embedding)


# speedup vs baseline: 1.2798x; 1.1185x over previous
import functools

import jax
import jax.numpy as jnp
from jax import lax
from jax.experimental import pallas as pl
from jax.experimental.pallas import tpu as pltpu
from jax.experimental.pallas import tpu_sc as plsc

B, L, D = 4096, 200, 128
LANES = 16
PE_ROWS = 216
SLOT_ROWS = 200
NSLOT = 4


def kernel(inputs, pos_embedding):
    info = plsc.get_sparse_core_info()
    nc, ns = info.num_cores, info.num_subcores
    nw = nc * ns
    rows = B // nw

    mesh = plsc.VectorSubcoreMesh(core_axis_name="c", subcore_axis_name="s")

    @functools.partial(
        pl.kernel,
        mesh=mesh,
        out_type=jax.ShapeDtypeStruct((B, L, D), jnp.float32),
        scratch_types=[
            pltpu.VMEM((PE_ROWS, D), jnp.float32),
            pltpu.VMEM((128,), jnp.int32),
            pltpu.VMEM((80,), jnp.int32),
            pltpu.VMEM_SHARED((ns, NSLOT, SLOT_ROWS, D), jnp.float32),
            pltpu.SemaphoreType.DMA,
            pltpu.SemaphoreType.DMA,
            pltpu.SemaphoreType.DMA,
            pltpu.SemaphoreType.DMA,
            pltpu.SemaphoreType.DMA,
            pltpu.SemaphoreType.DMA,
            pltpu.SemaphoreType.DMA,
            pltpu.SemaphoreType.DMA,
        ],
    )
    def sc_add(in_hbm, pe_hbm, out_hbm, pe_v, idx_a, idx_b, sp,
               la0, la1, la2, la3, st0, st1, st2, st3):
        c = lax.axis_index("c")
        s = lax.axis_index("s")
        wid = s * nc + c
        base = wid * rows
        lsems = (la0, la1, la2, la3)
        ssems = (st0, st1, st2, st3)

        pltpu.sync_copy(pe_hbm.at[pl.ds(0, 208)], pe_v.at[pl.ds(0, 208)])
        zero = jnp.zeros((LANES,), jnp.float32)
        for rr in range(201, PE_ROWS):
            for j in range(D // LANES):
                pe_v[rr, pl.ds(j * LANES, LANES)] = zero

        for k in range(8):
            idx_a[pl.ds(k * LANES, LANES)] = (
                lax.iota(jnp.int32, LANES) + k * LANES)
        for k in range(4):
            idx_b[pl.ds(k * LANES, LANES)] = (
                lax.iota(jnp.int32, LANES) + 128 + k * LANES)
        idx_b[pl.ds(4 * LANES, LANES)] = jnp.minimum(
            lax.iota(jnp.int32, LANES) + 192, 199)

        def data(k):
            return sp.at[s, k]

        def load(k, u):
            pltpu.async_copy(in_hbm.at[base + u], data(k), lsems[k])

        def wait_load(k, u):
            pltpu.make_async_copy(in_hbm.at[base + u], data(k),
                                  lsems[k]).wait()

        def store(k, u):
            pltpu.async_copy(data(k), out_hbm.at[base + u], ssems[k])

        def wait_store(k, u):
            pltpu.make_async_copy(data(k), out_hbm.at[base + u],
                                  ssems[k]).wait()

        def scatter_add(k):
            dst = sp.at[s, k]
            pltpu.sync_copy(pe_v.at[pl.ds(1, 128)], dst.at[idx_a], add=True)
            pltpu.sync_copy(pe_v.at[pl.ds(129, 80)], dst.at[idx_b], add=True)

        def process(k, u):
            wait_load(k, u)
            scatter_add(k)
            store(k, u)

        for k in range(NSLOT):
            load(k, k)
        for k in range(2):
            process(k, k)

        def round2(t2, carry):
            r = 4 * t2
            for k in range(2):
                wait_store(k, r + k)
                load(k, r + 4 + k)
            for k in range(2):
                process(2 + k, r + 2 + k)
            for k in range(2):
                process(k, r + 4 + k)
            for k in range(2):
                wait_store(2 + k, r + 2 + k)
                load(2 + k, r + 6 + k)
            return carry

        lax.fori_loop(0, (rows - 4) // 4, round2, 0)

        for k in range(2):
            process(2 + k, rows - 2 + k)
        for k in range(2):
            wait_store(k, rows - 4 + k)
        for k in range(2):
            wait_store(2 + k, rows - 2 + k)

    return sc_add(inputs, pos_embedding)


# speedup vs baseline: 1.2844x; 1.0036x over previous
import functools

import jax
import jax.numpy as jnp
from jax import lax
from jax.experimental import pallas as pl
from jax.experimental.pallas import tpu as pltpu
from jax.experimental.pallas import tpu_sc as plsc

B, L, D = 4096, 200, 128
LANES = 16
PE_ROWS = 216
SLOT_ROWS = 200
NSLOT = 4


def kernel(inputs, pos_embedding):
    info = plsc.get_sparse_core_info()
    nc, ns = info.num_cores, info.num_subcores
    nw = nc * ns
    rows = B // nw

    mesh = plsc.VectorSubcoreMesh(core_axis_name="c", subcore_axis_name="s")

    @functools.partial(
        pl.kernel,
        mesh=mesh,
        out_type=jax.ShapeDtypeStruct((B, L, D), jnp.float32),
        scratch_types=[
            pltpu.VMEM((PE_ROWS, D), jnp.float32),
            pltpu.VMEM((128,), jnp.int32),
            pltpu.VMEM((80,), jnp.int32),
            pltpu.VMEM_SHARED((ns, NSLOT, SLOT_ROWS, D), jnp.float32),
            pltpu.SemaphoreType.DMA,
            pltpu.SemaphoreType.DMA,
            pltpu.SemaphoreType.DMA,
            pltpu.SemaphoreType.DMA,
            pltpu.SemaphoreType.DMA,
            pltpu.SemaphoreType.DMA,
            pltpu.SemaphoreType.DMA,
            pltpu.SemaphoreType.DMA,
        ],
    )
    def sc_add(in_hbm, pe_hbm, out_hbm, pe_v, idx_a, idx_b, sp,
               la0, la1, la2, la3, st0, st1, st2, st3):
        c = lax.axis_index("c")
        s = lax.axis_index("s")
        wid = s * nc + c
        base = wid * rows
        lsems = (la0, la1, la2, la3)
        ssems = (st0, st1, st2, st3)

        pltpu.sync_copy(pe_hbm.at[pl.ds(0, 208)], pe_v.at[pl.ds(0, 208)])
        zero = jnp.zeros((LANES,), jnp.float32)
        for rr in range(201, PE_ROWS):
            for j in range(D // LANES):
                pe_v[rr, pl.ds(j * LANES, LANES)] = zero

        for k in range(8):
            idx_a[pl.ds(k * LANES, LANES)] = (
                lax.iota(jnp.int32, LANES) + k * LANES)
        for k in range(4):
            idx_b[pl.ds(k * LANES, LANES)] = (
                lax.iota(jnp.int32, LANES) + 128 + k * LANES)
        idx_b[pl.ds(4 * LANES, LANES)] = jnp.minimum(
            lax.iota(jnp.int32, LANES) + 192, 199)

        def data(k):
            return sp.at[s, k]

        def load(k, u):
            pltpu.async_copy(in_hbm.at[base + u], data(k), lsems[k])

        def wait_load(k, u):
            pltpu.make_async_copy(in_hbm.at[base + u], data(k),
                                  lsems[k]).wait()

        def store(k, u):
            pltpu.async_copy(data(k), out_hbm.at[base + u], ssems[k])

        def wait_store(k, u):
            pltpu.make_async_copy(data(k), out_hbm.at[base + u],
                                  ssems[k]).wait()

        def scatter_add(k):
            dst = sp.at[s, k]
            cp_a = pltpu.async_copy(
                pe_v.at[pl.ds(1, 128)], dst.at[idx_a], lsems[k], add=True)
            cp_b = pltpu.async_copy(
                pe_v.at[pl.ds(129, 80)], dst.at[idx_b], lsems[k], add=True)
            cp_a.wait()
            cp_b.wait()

        def process(k, u):
            wait_load(k, u)
            scatter_add(k)
            store(k, u)

        for k in range(NSLOT):
            load(k, k)
        for k in range(2):
            process(k, k)

        def round2(t2, carry):
            r = 4 * t2
            for k in range(2):
                wait_store(k, r + k)
                load(k, r + 4 + k)
            for k in range(2):
                process(2 + k, r + 2 + k)
            for k in range(2):
                process(k, r + 4 + k)
            for k in range(2):
                wait_store(2 + k, r + 2 + k)
                load(2 + k, r + 6 + k)
            return carry

        lax.fori_loop(0, (rows - 4) // 4, round2, 0)

        for k in range(2):
            process(2 + k, rows - 2 + k)
        for k in range(2):
            wait_store(k, rows - 4 + k)
        for k in range(2):
            wait_store(2 + k, rows - 2 + k)

    return sc_add(inputs, pos_embedding)
